# Initial kernel scaffold; baseline (speedup 1.0000x reference)
import jax
import jax.numpy as jnp
from jax.experimental import pallas as pl


def kernel(x, codebook):
    raise NotImplementedError("write your pallas kernel here")


# speedup vs baseline: 114.4272x; 114.4272x over previous
import functools

import jax
import jax.numpy as jnp
from jax import lax
from jax.experimental import pallas as pl
from jax.experimental.pallas import tpu as pltpu
from jax.experimental.pallas import tpu_sc as plsc

_D = 256
_K = 8192
_TB = 128
_KC = 512
_NTOK = 16 * 576

_NC = 2
_NS = 16
_NW = _NC * _NS
_BPW = _NTOK // _NW


def _snap_ids_kernel(x_ref, cbt_ref, ids_ref, c2_ref):
    @pl.when(pl.program_id(0) == 0)
    def _():
        for j in range(_K // _KC):
            c = cbt_ref[:, pl.ds(j * _KC, _KC)]
            c2_ref[:, pl.ds(j * _KC, _KC)] = jnp.sum(c * c, axis=0,
                                                     keepdims=True)

    x = x_ref[...]
    x2 = jnp.sum(x * x, axis=1, keepdims=True)

    m = jnp.full((_TB, 1), -jnp.inf, jnp.float32)
    idx = jnp.zeros((_TB, 1), jnp.int32)
    for j in range(_K // _KC):
        cb_chunk = cbt_ref[:, pl.ds(j * _KC, _KC)]
        xc = jnp.dot(x, cb_chunk, preferred_element_type=jnp.float32)
        c2 = c2_ref[:, pl.ds(j * _KC, _KC)]
        d2 = jnp.maximum(x2 + c2 - 2.0 * xc, 0.0)
        s = -jnp.sqrt(d2)
        cm = jnp.max(s, axis=1, keepdims=True)
        ii = lax.broadcasted_iota(jnp.int32, s.shape, 1) + j * _KC
        cidx = jnp.min(jnp.where(s == cm, ii, _K), axis=1, keepdims=True)
        better = cm > m
        idx = jnp.where(better, cidx, idx)
        m = jnp.where(better, cm, m)
    ids_ref[0, 0, :] = idx[:, 0]


def _compute_ids(x2d, cbt):
    nb = x2d.shape[0] // _TB
    ids = pl.pallas_call(
        _snap_ids_kernel,
        grid=(nb,),
        in_specs=[
            pl.BlockSpec((_TB, _D), lambda i: (i, 0)),
            pl.BlockSpec((_D, _K), lambda i: (0, 0)),
        ],
        out_specs=pl.BlockSpec((1, 1, _TB), lambda i: (i, 0, 0)),
        out_shape=jax.ShapeDtypeStruct((nb, 1, _TB), jnp.int32),
        scratch_shapes=[pltpu.VMEM((1, _K), jnp.float32)],
    )(x2d, cbt)
    return ids.reshape(-1)


def _sc_gather(codebook, ids):
    mesh = plsc.VectorSubcoreMesh(core_axis_name="c", subcore_axis_name="s")

    @functools.partial(
        pl.kernel,
        mesh=mesh,
        out_type=jax.ShapeDtypeStruct((_NTOK, _D), jnp.float32),
        scratch_types=[
            pltpu.VMEM((_BPW,), jnp.int32),
            pltpu.VMEM((_BPW, _D), jnp.float32),
            pltpu.SemaphoreType.DMA,
        ],
    )
    def gather(table_hbm, idx_hbm, out_hbm, idx_v, rows_v, sem):
        wid = lax.axis_index("s") * _NC + lax.axis_index("c")
        base = wid * _BPW
        pltpu.sync_copy(idx_hbm.at[pl.ds(base, _BPW)], idx_v)
        pltpu.async_copy(table_hbm.at[idx_v], rows_v, sem).wait()
        pltpu.sync_copy(rows_v, out_hbm.at[pl.ds(base, _BPW)])

    return gather(codebook, ids)


def kernel(x, codebook):
    b, s, d = x.shape
    x2d = x.reshape(b * s, d)
    cbt = codebook.T
    ids_flat = _compute_ids(x2d, cbt)
    out_flat = _sc_gather(codebook, ids_flat)
    return out_flat.reshape(b, s, d), ids_flat.reshape(b, s, 1)


# speedup vs baseline: 151.7008x; 1.3257x over previous
import functools

import jax
import jax.numpy as jnp
from jax import lax
from jax.experimental import pallas as pl
from jax.experimental.pallas import tpu as pltpu
from jax.experimental.pallas import tpu_sc as plsc

_D = 256
_K = 8192
_TB = 256
_KC = 512
_NTOK = 16 * 576

_NC = 2
_NS = 16
_NW = _NC * _NS
_BPW = _NTOK // _NW


def _snap_ids_kernel(x_ref, cbt_ref, x2_ref, c2_ref, ids_ref):
    x = x_ref[...]
    x2 = x2_ref[...]

    mq = jnp.full((_TB, _KC), jnp.inf, jnp.float32)
    mj = jnp.zeros((_TB, _KC), jnp.float32)
    for j in range(_K // _KC):
        cb_chunk = cbt_ref[:, pl.ds(j * _KC, _KC)]
        xc = jnp.dot(x, cb_chunk, preferred_element_type=jnp.float32)
        c2 = c2_ref[:, pl.ds(j * _KC, _KC)]
        q = jnp.sqrt(jnp.maximum(x2 + c2 - 2.0 * xc, 0.0))
        better = q < mq
        mj = jnp.where(better, jnp.float32(j), mj)
        mq = jnp.minimum(mq, q)

    lane = lax.broadcasted_iota(jnp.int32, (_TB, _KC), 1).astype(jnp.float32)
    gidx = mj * jnp.float32(_KC) + lane
    best = jnp.min(mq, axis=1, keepdims=True)
    big = jnp.float32(2.0 * _K)
    idxf = jnp.min(jnp.where(mq == best, gidx, big), axis=1)
    ids_ref[0, 0, :] = idxf.astype(jnp.int32)


def _compute_ids(x2d, cbt, x2, c2):
    nb = x2d.shape[0] // _TB
    ids = pl.pallas_call(
        _snap_ids_kernel,
        grid=(nb,),
        in_specs=[
            pl.BlockSpec((_TB, _D), lambda i: (i, 0)),
            pl.BlockSpec((_D, _K), lambda i: (0, 0)),
            pl.BlockSpec((_TB, 1), lambda i: (i, 0)),
            pl.BlockSpec((1, _K), lambda i: (0, 0)),
        ],
        out_specs=pl.BlockSpec((1, 1, _TB), lambda i: (i, 0, 0)),
        out_shape=jax.ShapeDtypeStruct((nb, 1, _TB), jnp.int32),
    )(x2d, cbt, x2, c2)
    return ids.reshape(-1)


def _sc_gather(codebook, ids):
    mesh = plsc.VectorSubcoreMesh(core_axis_name="c", subcore_axis_name="s")

    @functools.partial(
        pl.kernel,
        mesh=mesh,
        out_type=jax.ShapeDtypeStruct((_NTOK, _D), jnp.float32),
        scratch_types=[
            pltpu.VMEM((_BPW,), jnp.int32),
            pltpu.VMEM((_BPW, _D), jnp.float32),
            pltpu.SemaphoreType.DMA,
        ],
    )
    def gather(table_hbm, idx_hbm, out_hbm, idx_v, rows_v, sem):
        wid = lax.axis_index("s") * _NC + lax.axis_index("c")
        base = wid * _BPW
        pltpu.sync_copy(idx_hbm.at[pl.ds(base, _BPW)], idx_v)
        pltpu.async_copy(table_hbm.at[idx_v], rows_v, sem).wait()
        pltpu.sync_copy(rows_v, out_hbm.at[pl.ds(base, _BPW)])

    return gather(codebook, ids)


def kernel(x, codebook):
    b, s, d = x.shape
    x2d = x.reshape(b * s, d)
    cbt = codebook.T
    x2 = jnp.sum(x * x, axis=-1, keepdims=True).reshape(b * s, 1)
    c2 = jnp.sum(codebook * codebook, axis=-1).reshape(1, _K)
    ids_flat = _compute_ids(x2d, cbt, x2, c2)
    out_flat = _sc_gather(codebook, ids_flat)
    return out_flat.reshape(b, s, d), ids_flat.reshape(b, s, 1)


# speedup vs baseline: 158.3234x; 1.0437x over previous
import functools

import jax
import jax.numpy as jnp
from jax import lax
from jax.experimental import pallas as pl
from jax.experimental.pallas import tpu as pltpu
from jax.experimental.pallas import tpu_sc as plsc

_D = 256
_K = 8192
_TB = 512
_KC = 256
_NTOK = 16 * 576

_NC = 2
_NS = 16
_NW = _NC * _NS
_BPW = _NTOK // _NW


def _snap_ids_kernel(x_ref, cbt2_ref, ids_ref, c2_ref):
    @pl.when(pl.program_id(0) == 0)
    def _():
        for j in range(_K // _KC):
            c = 0.5 * cbt2_ref[:, pl.ds(j * _KC, _KC)]
            c2_ref[:, pl.ds(j * _KC, _KC)] = jnp.sum(c * c, axis=0,
                                                     keepdims=True)

    x = x_ref[...]
    x2 = jnp.sum(x * x, axis=1, keepdims=True)

    mq = jnp.full((_TB, _KC), jnp.inf, jnp.float32)
    mj = jnp.zeros((_TB, _KC), jnp.float32)
    for j in range(_K // _KC):
        xc2 = jnp.dot(x, cbt2_ref[:, pl.ds(j * _KC, _KC)],
                      preferred_element_type=jnp.float32)
        c2 = c2_ref[:, pl.ds(j * _KC, _KC)]
        q = jnp.sqrt(jnp.maximum(x2 + c2 - xc2, 0.0))
        better = q < mq
        mj = jnp.where(better, jnp.float32(j), mj)
        mq = jnp.where(better, q, mq)

    lane = lax.broadcasted_iota(jnp.int32, (_TB, _KC), 1).astype(jnp.float32)
    gidx = mj * jnp.float32(_KC) + lane
    best = jnp.min(mq, axis=1, keepdims=True)
    big = jnp.float32(2.0 * _K)
    idxf = jnp.min(jnp.where(mq == best, gidx, big), axis=1)
    ids_ref[0, 0, :] = idxf.astype(jnp.int32)


def _compute_ids(x2d, cbt2):
    nb = x2d.shape[0] // _TB
    ids = pl.pallas_call(
        _snap_ids_kernel,
        grid=(nb,),
        in_specs=[
            pl.BlockSpec((_TB, _D), lambda i: (i, 0)),
            pl.BlockSpec((_D, _K), lambda i: (0, 0)),
        ],
        out_specs=pl.BlockSpec((1, 1, _TB), lambda i: (i, 0, 0)),
        out_shape=jax.ShapeDtypeStruct((nb, 1, _TB), jnp.int32),
        scratch_shapes=[pltpu.VMEM((1, _K), jnp.float32)],
    )(x2d, cbt2)
    return ids.reshape(-1)


def _sc_gather(codebook, ids):
    mesh = plsc.VectorSubcoreMesh(core_axis_name="c", subcore_axis_name="s")

    @functools.partial(
        pl.kernel,
        mesh=mesh,
        out_type=jax.ShapeDtypeStruct((_NTOK, _D), jnp.float32),
        scratch_types=[
            pltpu.VMEM((_BPW,), jnp.int32),
            pltpu.VMEM((_BPW, _D), jnp.float32),
            pltpu.SemaphoreType.DMA,
        ],
    )
    def gather(table_hbm, idx_hbm, out_hbm, idx_v, rows_v, sem):
        wid = lax.axis_index("s") * _NC + lax.axis_index("c")
        base = wid * _BPW
        pltpu.sync_copy(idx_hbm.at[pl.ds(base, _BPW)], idx_v)
        pltpu.async_copy(table_hbm.at[idx_v], rows_v, sem).wait()
        pltpu.sync_copy(rows_v, out_hbm.at[pl.ds(base, _BPW)])

    return gather(codebook, ids)


def kernel(x, codebook):
    b, s, d = x.shape
    x2d = x.reshape(b * s, d)
    cbt2 = codebook.T * 2.0
    ids_flat = _compute_ids(x2d, cbt2)
    out_flat = _sc_gather(codebook, ids_flat)
    return out_flat.reshape(b, s, d), ids_flat.reshape(b, s, 1)


# speedup vs baseline: 163.7276x; 1.0341x over previous
import functools

import jax
import jax.numpy as jnp
from jax import lax
from jax.experimental import pallas as pl
from jax.experimental.pallas import tpu as pltpu
from jax.experimental.pallas import tpu_sc as plsc

_D = 256
_K = 8192
_TB = 768
_KC = 256
_NTOK = 16 * 576

_NC = 2
_NS = 16
_NW = _NC * _NS
_BPW = _NTOK // _NW


def _snap_ids_kernel(x_ref, cbt2_ref, ids_ref, c2_ref):
    @pl.when(pl.program_id(0) == 0)
    def _():
        for j in range(_K // _KC):
            c = 0.5 * cbt2_ref[:, pl.ds(j * _KC, _KC)]
            c2_ref[:, pl.ds(j * _KC, _KC)] = jnp.sum(c * c, axis=0,
                                                     keepdims=True)

    x = x_ref[...]
    x2 = jnp.sum(x * x, axis=1, keepdims=True)

    mq = jnp.full((_TB, _KC), jnp.inf, jnp.float32)
    mj = jnp.zeros((_TB, _KC), jnp.float32)
    for j in range(_K // _KC):
        xc2 = jnp.dot(x, cbt2_ref[:, pl.ds(j * _KC, _KC)],
                      preferred_element_type=jnp.float32)
        c2 = c2_ref[:, pl.ds(j * _KC, _KC)]
        q = jnp.sqrt(jnp.maximum(x2 + c2 - xc2, 0.0))
        better = q < mq
        mj = jnp.where(better, jnp.float32(j), mj)
        mq = jnp.where(better, q, mq)

    lane = lax.broadcasted_iota(jnp.int32, (_TB, _KC), 1).astype(jnp.float32)
    gidx = mj * jnp.float32(_KC) + lane
    best = jnp.min(mq, axis=1, keepdims=True)
    big = jnp.float32(2.0 * _K)
    idxf = jnp.min(jnp.where(mq == best, gidx, big), axis=1)
    ids_ref[0, 0, :] = idxf.astype(jnp.int32)


def _compute_ids(x2d, cbt2):
    nb = x2d.shape[0] // _TB
    ids = pl.pallas_call(
        _snap_ids_kernel,
        grid=(nb,),
        in_specs=[
            pl.BlockSpec((_TB, _D), lambda i: (i, 0)),
            pl.BlockSpec((_D, _K), lambda i: (0, 0)),
        ],
        out_specs=pl.BlockSpec((1, 1, _TB), lambda i: (i, 0, 0)),
        out_shape=jax.ShapeDtypeStruct((nb, 1, _TB), jnp.int32),
        scratch_shapes=[pltpu.VMEM((1, _K), jnp.float32)],
    )(x2d, cbt2)
    return ids.reshape(-1)


def _sc_gather(codebook, ids):
    mesh = plsc.VectorSubcoreMesh(core_axis_name="c", subcore_axis_name="s")

    @functools.partial(
        pl.kernel,
        mesh=mesh,
        out_type=jax.ShapeDtypeStruct((_NTOK, _D), jnp.float32),
        scratch_types=[
            pltpu.VMEM((_BPW,), jnp.int32),
            pltpu.VMEM((_BPW, _D), jnp.float32),
            pltpu.SemaphoreType.DMA,
        ],
    )
    def gather(table_hbm, idx_hbm, out_hbm, idx_v, rows_v, sem):
        wid = lax.axis_index("s") * _NC + lax.axis_index("c")
        base = wid * _BPW
        pltpu.sync_copy(idx_hbm.at[pl.ds(base, _BPW)], idx_v)
        pltpu.async_copy(table_hbm.at[idx_v], rows_v, sem).wait()
        pltpu.sync_copy(rows_v, out_hbm.at[pl.ds(base, _BPW)])

    return gather(codebook, ids)


def kernel(x, codebook):
    b, s, d = x.shape
    x2d = x.reshape(b * s, d)
    cbt2 = codebook.T * 2.0
    ids_flat = _compute_ids(x2d, cbt2)
    out_flat = _sc_gather(codebook, ids_flat)
    return out_flat.reshape(b, s, d), ids_flat.reshape(b, s, 1)
